# baseline (speedup 1.0000x reference)
import functools

import numpy as np
import jax
import jax.numpy as jnp
from jax import lax
from jax.experimental import pallas as pl
from jax.experimental.pallas import tpu as pltpu
from jax.experimental.pallas import tpu_sc as plsc

VOCAB = 100000
D_MODEL = 1024
MAX_LEN = 8192
B = 4
S = 4096

NC = 2
NS = 16
LANES = 16
NW = NC * NS
TOKENS = B * S
TPW = TOKENS // NW
TILE = 16
NTILES = TPW // TILE


def _positional_encoding(max_len, d_model):
    pos = np.arange(max_len, dtype=np.float32)[:, None]
    i = np.arange(0, d_model, 2, dtype=np.float32)
    div = np.power(10000.0, i / d_model)
    pe = np.zeros((max_len, d_model), dtype=np.float32)
    pe[:, 0::2] = np.sin(pos / div)
    pe[:, 1::2] = np.cos(pos / div)
    return pe


_PE = _positional_encoding(MAX_LEN, D_MODEL)[:S]


_MESH = plsc.VectorSubcoreMesh(core_axis_name="c", subcore_axis_name="s")


@functools.partial(
    pl.kernel,
    mesh=_MESH,
    out_type=jax.ShapeDtypeStruct((TOKENS, D_MODEL), jnp.float32),
    scratch_types=[
        pltpu.VMEM((TPW,), jnp.int32),
        pltpu.VMEM((TILE, D_MODEL), jnp.float32),
        pltpu.VMEM((TILE, D_MODEL), jnp.float32),
        pltpu.SemaphoreType.DMA,
        pltpu.SemaphoreType.DMA,
    ],
)
def _embed(idx_hbm, table_hbm, pe_hbm, out_hbm, idx_v, rows_v, pe_v, sem_g, sem_p):
    wid = lax.axis_index("s") * NC + lax.axis_index("c")
    base = wid * TPW
    pe0 = base % S

    pltpu.sync_copy(idx_hbm.at[pl.ds(base, TPW)], idx_v)

    def tile_step(g, carry):
        row0 = base + g * TILE
        gather = pltpu.async_copy(
            table_hbm.at[idx_v.at[pl.ds(g * TILE, TILE)]], rows_v, sem_g)
        pe_cp = pltpu.async_copy(
            pe_hbm.at[pl.ds(pe0 + g * TILE, TILE)], pe_v, sem_p)
        gather.wait()
        pe_cp.wait()

        def row_step(r, c0):
            def col_step(c, c1):
                sl = pl.ds(c * LANES, LANES)
                plsc.addupdate(rows_v.at[r, sl], pe_v[r, sl])
                return c1
            return lax.fori_loop(0, D_MODEL // LANES, col_step, c0)

        lax.fori_loop(0, TILE, row_step, 0)
        pltpu.sync_copy(rows_v, out_hbm.at[pl.ds(row0, TILE)])
        return carry

    lax.fori_loop(0, NTILES, tile_step, 0)


def kernel(x, table):
    idx = x.reshape(-1).astype(jnp.int32)
    pe = jnp.asarray(_PE)
    out = _embed(idx, table, pe)
    return out.reshape(B, S, D_MODEL)


# speedup vs baseline: 1.3091x; 1.3091x over previous
import functools

import numpy as np
import jax
import jax.numpy as jnp
from jax import lax
from jax.experimental import pallas as pl
from jax.experimental.pallas import tpu as pltpu
from jax.experimental.pallas import tpu_sc as plsc

VOCAB = 100000
D_MODEL = 1024
MAX_LEN = 8192
B = 4
S = 4096

NC = 2
NS = 16
LANES = 16
NW = NC * NS
PPW = S // NW
TILE = 8
NT = PPW // TILE


def _positional_encoding(max_len, d_model):
    pos = np.arange(max_len, dtype=np.float32)[:, None]
    i = np.arange(0, d_model, 2, dtype=np.float32)
    div = np.power(10000.0, i / d_model)
    pe = np.zeros((max_len, d_model), dtype=np.float32)
    pe[:, 0::2] = np.sin(pos / div)
    pe[:, 1::2] = np.cos(pos / div)
    return pe


_PE = _positional_encoding(MAX_LEN, D_MODEL)[:S]


_MESH = plsc.VectorSubcoreMesh(core_axis_name="c", subcore_axis_name="s")


@functools.partial(
    pl.kernel,
    mesh=_MESH,
    out_type=jax.ShapeDtypeStruct((B, S, D_MODEL), jnp.float32),
    scratch_types=[
        pltpu.VMEM((B, PPW), jnp.int32),
        pltpu.VMEM((B, TILE, D_MODEL), jnp.float32),
        pltpu.VMEM((B, TILE, D_MODEL), jnp.float32),
        pltpu.VMEM((TILE, D_MODEL), jnp.float32),
        pltpu.VMEM((TILE, D_MODEL), jnp.float32),
        pltpu.SemaphoreType.DMA,
        pltpu.SemaphoreType.DMA,
        pltpu.SemaphoreType.DMA,
        pltpu.SemaphoreType.DMA,
    ],
)
def _embed(x_hbm, table_hbm, pe_hbm, out_hbm,
           idx_v, rows_a, rows_b, pe_a, pe_b, g_sem_a, g_sem_b,
           o_sem_a, o_sem_b):
    wid = lax.axis_index("s") * NC + lax.axis_index("c")
    p0 = wid * PPW

    pltpu.sync_copy(x_hbm.at[:, pl.ds(p0, PPW)], idx_v)

    def issue(t, rows_v, pe_v, g_sem):
        pltpu.async_copy(pe_hbm.at[pl.ds(p0 + t * TILE, TILE)], pe_v, g_sem)
        for b in range(B):
            pltpu.async_copy(
                table_hbm.at[idx_v.at[b, pl.ds(t * TILE, TILE)]],
                rows_v.at[b], g_sem)

    def drain_gathers(rows_v, pe_v, g_sem):
        pltpu.make_async_copy(pe_hbm.at[pl.ds(0, TILE)], pe_v, g_sem).wait()
        for b in range(B):
            pltpu.make_async_copy(
                table_hbm.at[pl.ds(0, TILE)], rows_v.at[b], g_sem).wait()

    def process(t, rows_v, pe_v, g_sem, o_sem):
        drain_gathers(rows_v, pe_v, g_sem)

        def row_step(r, carry):
            for b in range(B):
                for c in range(D_MODEL // LANES):
                    sl = pl.ds(c * LANES, LANES)
                    plsc.addupdate(rows_v.at[b, r, sl], pe_v[r, sl])
            return carry

        lax.fori_loop(0, TILE, row_step, 0)
        pltpu.async_copy(
            rows_v, out_hbm.at[:, pl.ds(p0 + t * TILE, TILE), :], o_sem)

    def drain_out(rows_v, o_sem):
        pltpu.make_async_copy(
            rows_v, out_hbm.at[:, pl.ds(0, TILE), :], o_sem).wait()

    issue(0, rows_a, pe_a, g_sem_a)
    issue(1, rows_b, pe_b, g_sem_b)

    def pair_step(t2, carry):
        t = t2 * 2
        process(t, rows_a, pe_a, g_sem_a, o_sem_a)

        @pl.when(t + 2 < NT)
        def _():
            drain_out(rows_a, o_sem_a)
            issue(t + 2, rows_a, pe_a, g_sem_a)

        process(t + 1, rows_b, pe_b, g_sem_b, o_sem_b)

        @pl.when(t + 3 < NT)
        def _():
            drain_out(rows_b, o_sem_b)
            issue(t + 3, rows_b, pe_b, g_sem_b)

        return carry

    lax.fori_loop(0, NT // 2, pair_step, 0)
    drain_out(rows_a, o_sem_a)
    drain_out(rows_b, o_sem_b)


def kernel(x, table):
    pe = jnp.asarray(_PE)
    return _embed(x.astype(jnp.int32), table, pe)


# speedup vs baseline: 2.9554x; 2.2576x over previous
import functools

import numpy as np
import jax
import jax.numpy as jnp
from jax import lax
from jax.experimental import pallas as pl
from jax.experimental.pallas import tpu as pltpu
from jax.experimental.pallas import tpu_sc as plsc

VOCAB = 100000
D_MODEL = 1024
MAX_LEN = 8192
B = 4
S = 4096

NC = 2
NS = 16
LANES = 16
NW = NC * NS
PPW = S // NW
TILE = 8
BT = B * TILE
NT = PPW // TILE
NBUF = 3


def _positional_encoding(max_len, d_model):
    pos = np.arange(max_len, dtype=np.float32)[:, None]
    i = np.arange(0, d_model, 2, dtype=np.float32)
    div = np.power(10000.0, i / d_model)
    pe = np.zeros((max_len, d_model), dtype=np.float32)
    pe[:, 0::2] = np.sin(pos / div)
    pe[:, 1::2] = np.cos(pos / div)
    return pe


_PE = _positional_encoding(MAX_LEN, D_MODEL)[:S]


_MESH = plsc.VectorSubcoreMesh(core_axis_name="c", subcore_axis_name="s")


@functools.partial(
    pl.kernel,
    mesh=_MESH,
    out_type=jax.ShapeDtypeStruct((B, S, D_MODEL), jnp.float32),
    scratch_types=(
        [pltpu.VMEM((NT, BT), jnp.int32)]
        + [pltpu.VMEM((BT, D_MODEL), jnp.float32) for _ in range(NBUF)]
        + [pltpu.VMEM((TILE, D_MODEL), jnp.float32) for _ in range(NBUF)]
        + [pltpu.SemaphoreType.DMA for _ in range(NBUF)]
        + [pltpu.SemaphoreType.DMA for _ in range(NBUF)]
    ),
)
def _embed(idx_hbm, table_hbm, pe_hbm, out_hbm,
           idx_v, rows0, rows1, rows2, pe0, pe1, pe2,
           g0, g1, g2, o0, o1, o2):
    rows_s = (rows0, rows1, rows2)
    pe_s = (pe0, pe1, pe2)
    g_s = (g0, g1, g2)
    o_s = (o0, o1, o2)

    wid = lax.axis_index("s") * NC + lax.axis_index("c")
    p0 = wid * PPW

    pltpu.sync_copy(idx_hbm.at[wid], idx_v)

    def issue(u, k):
        pltpu.async_copy(pe_hbm.at[pl.ds(p0 + u * TILE, TILE)], pe_s[k], g_s[k])
        pltpu.async_copy(table_hbm.at[idx_v.at[u]], rows_s[k], g_s[k])

    def drain_gathers(k):
        pltpu.make_async_copy(pe_hbm.at[pl.ds(0, TILE)], pe_s[k], g_s[k]).wait()
        pltpu.make_async_copy(
            table_hbm.at[pl.ds(0, BT)], rows_s[k], g_s[k]).wait()

    def add_pe(k):
        rows_v, pe_v = rows_s[k], pe_s[k]

        def row_step(r, carry):
            for c in range(D_MODEL // LANES):
                sl = pl.ds(c * LANES, LANES)
                v = pe_v[r, sl]
                for b in range(B):
                    plsc.addupdate(rows_v.at[b * TILE + r, sl], v)
            return carry

        lax.fori_loop(0, TILE, row_step, 0)

    def issue_out(u, k):
        for b in range(B):
            pltpu.async_copy(
                rows_s[k].at[pl.ds(b * TILE, TILE)],
                out_hbm.at[b, pl.ds(p0 + u * TILE, TILE), :], o_s[k])

    def drain_out(k):
        for b in range(B):
            pltpu.make_async_copy(
                rows_s[k].at[pl.ds(b * TILE, TILE)],
                out_hbm.at[0, pl.ds(0, TILE), :], o_s[k]).wait()

    def tile_body(u, k, first, last_issued):
        drain_gathers(k)
        add_pe(k)
        if not first:
            drain_out((k + NBUF - 1) % NBUF)

            @pl.when(u + 2 < NT)
            def _():
                issue(u + 2, (k + NBUF - 1) % NBUF)
        else:
            issue(u + 2, (k + NBUF - 1) % NBUF)
        issue_out(u, k)

    issue(0, 0)
    issue(1, 1)
    tile_body(0, 0, True, None)

    def triple_step(t3, carry):
        u = 1 + t3 * 3
        tile_body(u, 1, False, None)
        tile_body(u + 1, 2, False, None)
        tile_body(u + 2, 0, False, None)
        return carry

    lax.fori_loop(0, (NT - 1) // 3, triple_step, 0)

    drain_out(0)


def kernel(x, table):
    idx3 = (x.astype(jnp.int32)
            .reshape(B, NW, NT, TILE)
            .transpose(1, 2, 0, 3)
            .reshape(NW, NT, BT))
    pe = jnp.asarray(_PE)
    return _embed(idx3, table, pe)


# speedup vs baseline: 2.9648x; 1.0032x over previous
import functools

import numpy as np
import jax
import jax.numpy as jnp
from jax import lax
from jax.experimental import pallas as pl
from jax.experimental.pallas import tpu as pltpu
from jax.experimental.pallas import tpu_sc as plsc

VOCAB = 100000
D_MODEL = 1024
MAX_LEN = 8192
B = 4
S = 4096

NC = 2
NS = 16
LANES = 16
NW = NC * NS
PPW = S // NW
CHUNK = 16
NCH = PPW // CHUNK


def _positional_encoding(max_len, d_model):
    pos = np.arange(max_len, dtype=np.float32)[:, None]
    i = np.arange(0, d_model, 2, dtype=np.float32)
    div = np.power(10000.0, i / d_model)
    pe = np.zeros((max_len, d_model), dtype=np.float32)
    pe[:, 0::2] = np.sin(pos / div)
    pe[:, 1::2] = np.cos(pos / div)
    return pe


_PE = _positional_encoding(MAX_LEN, D_MODEL)[:S]


_MESH = plsc.VectorSubcoreMesh(core_axis_name="c", subcore_axis_name="s")


@functools.partial(
    pl.kernel,
    mesh=_MESH,
    out_type=jax.ShapeDtypeStruct((B, S, D_MODEL), jnp.float32),
    scratch_types=(
        [pltpu.VMEM((B, PPW), jnp.int32)]
        + [pltpu.VMEM((CHUNK, D_MODEL), jnp.float32) for _ in range(B)]
        + [pltpu.VMEM((CHUNK, D_MODEL), jnp.float32) for _ in range(2)]
        + [pltpu.SemaphoreType.DMA for _ in range(B)]
        + [pltpu.SemaphoreType.DMA for _ in range(B)]
        + [pltpu.SemaphoreType.DMA for _ in range(2)]
    ),
)
def _embed(x_hbm, table_hbm, pe_hbm, out_hbm,
           idx_v, rows0, rows1, rows2, rows3, peb0, peb1,
           g0, g1, g2, g3, o0, o1, o2, o3, ps0, ps1):
    rows_s = (rows0, rows1, rows2, rows3)
    pe_s = (peb0, peb1)
    g_s = (g0, g1, g2, g3)
    o_s = (o0, o1, o2, o3)

    wid = lax.axis_index("s") * NC + lax.axis_index("c")
    p0 = wid * PPW

    for b in range(B):
        pltpu.sync_copy(x_hbm.at[b, pl.ds(p0, PPW)], idx_v.at[b])

    def issue_gather(c, b):
        pltpu.async_copy(
            table_hbm.at[idx_v.at[b, pl.ds(c * CHUNK, CHUNK)]],
            rows_s[b], g_s[b])

    def drain_gather(b):
        pltpu.make_async_copy(
            table_hbm.at[pl.ds(0, CHUNK)], rows_s[b], g_s[b]).wait()

    def issue_pe(c, j):
        pltpu.async_copy(
            pe_hbm.at[pl.ds(p0 + c * CHUNK, CHUNK)], pe_s[j], ps0 if j == 0 else ps1)

    def drain_pe(j):
        pltpu.make_async_copy(
            pe_hbm.at[pl.ds(0, CHUNK)], pe_s[j], ps0 if j == 0 else ps1).wait()

    def add_pe(b, j):
        rows_v, pe_v = rows_s[b], pe_s[j]

        def row_step(r, carry):
            for cc in range(D_MODEL // LANES):
                sl = pl.ds(cc * LANES, LANES)
                plsc.addupdate(rows_v.at[r, sl], pe_v[r, sl])
            return carry

        lax.fori_loop(0, CHUNK, row_step, 0)

    def issue_out(c, b):
        pltpu.async_copy(
            rows_s[b], out_hbm.at[b, pl.ds(p0 + c * CHUNK, CHUNK), :], o_s[b])

    def drain_out(b):
        pltpu.make_async_copy(
            rows_s[b], out_hbm.at[0, pl.ds(0, CHUNK), :], o_s[b]).wait()

    def step(c, b, j, first=False, guard=False):
        drain_gather(b)
        if b == 0:
            drain_pe(j)
        add_pe(b, j)
        if not first:
            drain_out((b + 3) % B)
        cn = c if b == 0 else c + 1
        if guard:
            @pl.when(cn < NCH)
            def _():
                issue_gather(cn, (b + 3) % B)
        else:
            issue_gather(cn, (b + 3) % B)
        issue_out(c, b)

    issue_pe(0, 0)
    issue_pe(1, 1)
    issue_gather(0, 0)
    issue_gather(0, 1)
    issue_gather(0, 2)

    step(0, 0, 0, first=True)
    step(0, 1, 0)
    step(0, 2, 0)
    step(0, 3, 0)
    issue_pe(2, 0)
    step(1, 0, 1)
    step(1, 1, 1)
    step(1, 2, 1)
    step(1, 3, 1)
    issue_pe(3, 1)

    def pair_body(c2, carry):
        c = c2 * 2
        step(c, 0, 0)
        step(c, 1, 0)
        step(c, 2, 0, guard=True)
        step(c, 3, 0, guard=True)

        @pl.when(c + 2 < NCH)
        def _():
            issue_pe(c + 2, 0)

        step(c + 1, 0, 1, guard=True)
        step(c + 1, 1, 1, guard=True)
        step(c + 1, 2, 1, guard=True)
        step(c + 1, 3, 1, guard=True)

        @pl.when(c + 3 < NCH)
        def _():
            issue_pe(c + 3, 1)

        return carry

    lax.fori_loop(1, NCH // 2, pair_body, 0)

    drain_out(3)


def kernel(x, table):
    pe = jnp.asarray(_PE)
    return _embed(x.astype(jnp.int32), table, pe)
